# baseline (speedup 1.0000x reference)
import functools

import jax
import jax.numpy as jnp
from jax import lax
from jax.experimental import pallas as pl
from jax.experimental.pallas import tpu as pltpu
from jax.experimental.pallas import tpu_sc as plsc

N = 10000
E = 320000
D = 128

NC = 2
NS = 16
NW = NC * NS
C = 128
NCH = 80
EPAD = NW * NCH * C
NP = 10240
RPT = NP // NS
BT = 1024

_mesh = plsc.VectorSubcoreMesh(core_axis_name="c", subcore_axis_name="s")


def _make_sc(with_counts: bool):
  out_type = [jax.ShapeDtypeStruct((NC, NP, D), jnp.float32)]
  scratch = [
      pltpu.VMEM_SHARED((NP, D), jnp.bfloat16),
      pltpu.VMEM((NCH, C), jnp.int32),
      pltpu.VMEM((NCH, C), jnp.int32),
      pltpu.VMEM((C, D), jnp.bfloat16),
      pltpu.VMEM((C, D), jnp.bfloat16),
      pltpu.VMEM((C, D), jnp.float32),
      pltpu.VMEM((C, D), jnp.float32),
      pltpu.SemaphoreType.DMA,
      pltpu.SemaphoreType.DMA,
      pltpu.SemaphoreType.DMA,
      pltpu.SemaphoreType.DMA,
      pltpu.SemaphoreType.DMA,
      pltpu.SemaphoreType.DMA,
      pltpu.SemaphoreType.DMA,
      pltpu.SemaphoreType.DMA,
  ]
  if with_counts:
    out_type.append(jax.ShapeDtypeStruct((NC * NP,), jnp.float32))
    scratch += [
        pltpu.VMEM_SHARED((NP,), jnp.float32),
        pltpu.VMEM((C,), jnp.float32),
        pltpu.VMEM((RPT,), jnp.float32),
        pltpu.SemaphoreType.DMA,
        pltpu.SemaphoreType.DMA,
    ]

  def body(table, src_i, dst_i, acc_out, cnt_out, acc_sh, src_v, dst_v,
           rows0, rows1, fbuf, fbuf1, sem_i0, sem_i1, sem_o0, sem_o1,
           sem_g0, sem_g1, sem_s0, sem_s1,
           cnt_sh=None, ones_v=None, cnt_v=None, sem_c0=None, sem_c1=None):
    c = lax.axis_index("c")
    s = lax.axis_index("s")
    w = c * NS + s
    zero16 = jnp.zeros((16,), jnp.float32)
    zero32 = jnp.zeros((32,), jnp.bfloat16)

    @pl.loop(0, C)
    def _(i):
      for k in range(D // 32):
        rows0[i, pl.ds(32 * k, 32)] = zero32

    @pl.loop(0, RPT // C)
    def _(k):
      pltpu.sync_copy(rows0, acc_sh.at[pl.ds(s * RPT + k * C, C)])

    pltpu.sync_copy(src_i.at[w], src_v)
    pltpu.sync_copy(dst_i.at[w], dst_v)
    if with_counts:
      @pl.loop(0, RPT // 16)
      def _(i):
        cnt_v[pl.ds(16 * i, 16)] = zero16
      pltpu.sync_copy(cnt_v, cnt_sh.at[pl.ds(s * RPT, RPT)])
      one16 = jnp.ones((16,), jnp.float32)
      for k in range(C // 16):
        ones_v[pl.ds(16 * k, 16)] = one16
    plsc.subcore_barrier()

    pltpu.async_copy(table.at[src_v.at[0]], rows0, sem_g0)
    pltpu.async_copy(table.at[src_v.at[1]], rows1, sem_g1)

    @pl.loop(0, NCH, step=2)
    def _(j):
      pltpu.make_async_copy(table.at[src_v.at[j]], rows0, sem_g0).wait()
      pltpu.sync_copy(rows0, acc_sh.at[dst_v.at[j]], add=True)
      if with_counts:
        pltpu.sync_copy(ones_v, cnt_sh.at[dst_v.at[j]], add=True)
      pltpu.async_copy(table.at[src_v.at[(j + 2) % NCH]], rows0, sem_g0)

      j1 = j + 1
      pltpu.make_async_copy(table.at[src_v.at[j1]], rows1, sem_g1).wait()
      pltpu.sync_copy(rows1, acc_sh.at[dst_v.at[j1]], add=True)
      if with_counts:
        pltpu.sync_copy(ones_v, cnt_sh.at[dst_v.at[j1]], add=True)
      pltpu.async_copy(table.at[src_v.at[(j1 + 2) % NCH]], rows1, sem_g1)

    pltpu.make_async_copy(table.at[src_v.at[0]], rows0, sem_g0).wait()
    pltpu.make_async_copy(table.at[src_v.at[1]], rows1, sem_g1).wait()
    plsc.subcore_barrier()

    ev_idx = 2 * lax.iota(jnp.int32, 16)
    NB = RPT // C
    rbufs = (rows0, rows1)
    fbufs = (fbuf, fbuf1)
    sin = (sem_i0, sem_i1)
    sout = (sem_o0, sem_o1)

    def blk(k):
      return acc_sh.at[pl.ds(s * RPT + k * C, C)]

    def oblk(k):
      return acc_out.at[c].at[pl.ds(s * RPT + k * C, C)]

    pltpu.async_copy(blk(0), rows0, sem_i0)
    for k in range(NB):
      b = k % 2
      pltpu.make_async_copy(blk(k), rbufs[b], sin[b]).wait()
      if k + 1 < NB:
        pltpu.async_copy(blk(k + 1), rbufs[1 - b], sin[1 - b])
      if k >= 2:
        pltpu.make_async_copy(fbufs[b], oblk(k - 2), sout[b]).wait()

      @pl.loop(0, C)
      def _(i):
        for g in range(D // 32):
          v = rbufs[b][i, pl.ds(32 * g, 32)]
          a, bb = plsc.unpack(v, format=plsc.PackFormat.INTERLEAVED)
          plsc.store_scatter(fbufs[b].at[i], [32 * g + ev_idx], a)
          plsc.store_scatter(fbufs[b].at[i], [32 * g + 1 + ev_idx], bb)

      pltpu.async_copy(fbufs[b], oblk(k), sout[b])
    pltpu.make_async_copy(fbuf, oblk(NB - 2), sem_o0).wait()
    pltpu.make_async_copy(fbuf1, oblk(NB - 1), sem_o1).wait()
    if with_counts:
      pltpu.sync_copy(cnt_sh.at[pl.ds(s * RPT, RPT)], cnt_v)
      pltpu.sync_copy(cnt_v, cnt_out.at[pl.ds(c * NP + s * RPT, RPT)])

  if with_counts:
    def body_wc(table, src_i, dst_i, acc_out, cnt_out, acc_sh, src_v,
                dst_v, rows0, rows1, fbuf, fbuf1, sem_i0, sem_i1,
                sem_o0, sem_o1, sem_g0, sem_g1, sem_s0, sem_s1,
                cnt_sh, ones_v, cnt_v, sem_c0, sem_c1):
      body(table, src_i, dst_i, acc_out, cnt_out, acc_sh, src_v, dst_v,
           rows0, rows1, fbuf, fbuf1, sem_i0, sem_i1, sem_o0, sem_o1,
           sem_g0, sem_g1, sem_s0, sem_s1,
           cnt_sh, ones_v, cnt_v, sem_c0, sem_c1)
    fn = body_wc
  else:
    def body_nc(table, src_i, dst_i, acc_out, acc_sh, src_v, dst_v,
                rows0, rows1, fbuf, fbuf1, sem_i0, sem_i1, sem_o0, sem_o1,
                sem_g0, sem_g1, sem_s0, sem_s1):
      body(table, src_i, dst_i, acc_out, None, acc_sh, src_v, dst_v,
           rows0, rows1, fbuf, fbuf1, sem_i0, sem_i1, sem_o0, sem_o1,
           sem_g0, sem_g1, sem_s0, sem_s1)
    fn = body_nc

  return pl.kernel(
      fn, out_type=out_type, mesh=_mesh, scratch_types=scratch,
      compiler_params=pltpu.CompilerParams(
          use_tc_tiling_on_sc=False, needs_layout_passes=False),
      name="sc_agg_cnt" if with_counts else "sc_agg")


_sc_agg_counts = _make_sc(True)
_sc_agg = _make_sc(False)


def _make_combine(out_dtype):
  def body(acc_ref, cnt_ref, h_ref, wl_ref, wr_ref, b_ref, out_ref):
    agg = acc_ref[0] + acc_ref[1]
    cnt = jnp.sum(cnt_ref[...], axis=0)[:, None]
    mean = agg * (1.0 / jnp.maximum(cnt, 1.0))
    dn = (((1,), (1,)), ((), ()))
    out = (
        lax.dot_general(mean, wl_ref[...], dn,
                        preferred_element_type=jnp.float32)
        + lax.dot_general(h_ref[...].astype(jnp.float32), wr_ref[...], dn,
                          preferred_element_type=jnp.float32)
        + b_ref[...])
    out_ref[...] = out.astype(out_dtype)

  return pl.pallas_call(
      body,
      grid=(NP // BT,),
      in_specs=[
          pl.BlockSpec((NC, BT, D), lambda i: (0, i, 0)),
          pl.BlockSpec((NC, BT), lambda i: (0, i)),
          pl.BlockSpec((BT, D), lambda i: (i, 0)),
          pl.BlockSpec((D, D), lambda i: (0, 0)),
          pl.BlockSpec((D, D), lambda i: (0, 0)),
          pl.BlockSpec((1, D), lambda i: (0, 0)),
      ],
      out_specs=pl.BlockSpec((BT, D), lambda i: (i, 0)),
      out_shape=jax.ShapeDtypeStruct((N, D), out_dtype),
  )


_tc_combine_mid = _make_combine(jnp.bfloat16)
_tc_combine_out = _make_combine(jnp.float32)


@jax.jit
def kernel(x, edge_index, W_l0, b_l0, W_r0, W_l1, b_l1, W_r1):
  pad = EPAD - E
  src = jnp.pad(edge_index[0], (0, pad)).reshape(NW, NCH, C)
  dst = jnp.pad(edge_index[1], (0, pad),
                constant_values=N).reshape(NW, NCH, C)
  x_bf = x.astype(jnp.bfloat16)
  acc1, cnt1 = _sc_agg_counts(x_bf, src, dst)
  cnt1 = cnt1.reshape(NC, NP)
  h1 = _tc_combine_mid(acc1, cnt1, x_bf, W_l0, W_r0, b_l0.reshape(1, D))
  (acc2,) = _sc_agg(h1, src, dst)
  out = _tc_combine_out(acc2, cnt1, h1, W_l1, W_r1, b_l1.reshape(1, D))
  return out


# speedup vs baseline: 2.4949x; 2.4949x over previous
import functools

import jax
import jax.numpy as jnp
from jax import lax
from jax.experimental import pallas as pl
from jax.experimental.pallas import tpu as pltpu
from jax.experimental.pallas import tpu_sc as plsc

N = 10000
E = 320000
D = 128

NC = 2
NS = 16
NW = NC * NS
C = 128
NCH = 80
EPAD = NW * NCH * C
NP = 10240
RPT = NP // NS
BT = 1024

_mesh = plsc.VectorSubcoreMesh(core_axis_name="c", subcore_axis_name="s")


def _make_sc(with_counts: bool):
  out_type = [jax.ShapeDtypeStruct((NC, NP, D), jnp.float32)]
  scratch = [
      pltpu.VMEM_SHARED((NP, D), jnp.bfloat16),
      pltpu.VMEM((NCH, C), jnp.int32),
      pltpu.VMEM((NCH, C), jnp.int32),
      pltpu.VMEM((C, D), jnp.bfloat16),
      pltpu.VMEM((C, D), jnp.bfloat16),
      pltpu.VMEM((C, D), jnp.float32),
      pltpu.VMEM((C, D), jnp.float32),
      pltpu.SemaphoreType.DMA,
      pltpu.SemaphoreType.DMA,
      pltpu.SemaphoreType.DMA,
      pltpu.SemaphoreType.DMA,
      pltpu.SemaphoreType.DMA,
      pltpu.SemaphoreType.DMA,
      pltpu.SemaphoreType.DMA,
      pltpu.SemaphoreType.DMA,
  ]
  if with_counts:
    out_type.append(jax.ShapeDtypeStruct((NC * NP,), jnp.float32))
    scratch += [
        pltpu.VMEM_SHARED((NP,), jnp.float32),
        pltpu.VMEM((C,), jnp.float32),
        pltpu.VMEM((RPT,), jnp.float32),
        pltpu.SemaphoreType.DMA,
        pltpu.SemaphoreType.DMA,
    ]

  def body(table, src_i, dst_i, acc_out, cnt_out, acc_sh, src_v, dst_v,
           rows0, rows1, fbuf, fbuf1, sem_i0, sem_i1, sem_o0, sem_o1,
           sem_g0, sem_g1, sem_s0, sem_s1,
           cnt_sh=None, ones_v=None, cnt_v=None, sem_c0=None, sem_c1=None):
    c = lax.axis_index("c")
    s = lax.axis_index("s")
    w = c * NS + s
    zero16 = jnp.zeros((16,), jnp.float32)
    zero32 = jnp.zeros((32,), jnp.bfloat16)

    @pl.loop(0, C)
    def _(i):
      for k in range(D // 32):
        rows0[i, pl.ds(32 * k, 32)] = zero32

    @pl.loop(0, RPT // C)
    def _(k):
      pltpu.sync_copy(rows0, acc_sh.at[pl.ds(s * RPT + k * C, C)])

    pltpu.sync_copy(src_i.at[w], src_v)
    pltpu.sync_copy(dst_i.at[w], dst_v)
    if with_counts:
      @pl.loop(0, RPT // 16)
      def _(i):
        cnt_v[pl.ds(16 * i, 16)] = zero16
      pltpu.sync_copy(cnt_v, cnt_sh.at[pl.ds(s * RPT, RPT)])
      one16 = jnp.ones((16,), jnp.float32)
      for k in range(C // 16):
        ones_v[pl.ds(16 * k, 16)] = one16
    plsc.subcore_barrier()

    pltpu.async_copy(table.at[src_v.at[0]], rows0, sem_g0)
    pltpu.async_copy(table.at[src_v.at[1]], rows1, sem_g1)

    @pl.loop(0, NCH, step=2)
    def _(j):
      pltpu.make_async_copy(table.at[src_v.at[j]], rows0, sem_g0).wait()
      pltpu.sync_copy(rows0, acc_sh.at[dst_v.at[j]], add=True)
      if with_counts:
        pltpu.sync_copy(ones_v, cnt_sh.at[dst_v.at[j]], add=True)
      pltpu.async_copy(table.at[src_v.at[(j + 2) % NCH]], rows0, sem_g0)

      j1 = j + 1
      pltpu.make_async_copy(table.at[src_v.at[j1]], rows1, sem_g1).wait()
      pltpu.sync_copy(rows1, acc_sh.at[dst_v.at[j1]], add=True)
      if with_counts:
        pltpu.sync_copy(ones_v, cnt_sh.at[dst_v.at[j1]], add=True)
      pltpu.async_copy(table.at[src_v.at[(j1 + 2) % NCH]], rows1, sem_g1)

    pltpu.make_async_copy(table.at[src_v.at[0]], rows0, sem_g0).wait()
    pltpu.make_async_copy(table.at[src_v.at[1]], rows1, sem_g1).wait()
    plsc.subcore_barrier()

    ev_idx = 2 * lax.iota(jnp.int32, 16)
    NB = RPT // C
    rbufs = (rows0, rows1)
    fbufs = (fbuf, fbuf1)
    sin = (sem_i0, sem_i1)
    sout = (sem_o0, sem_o1)

    def blk(k):
      return acc_sh.at[pl.ds(s * RPT + k * C, C)]

    def oblk(k):
      return acc_out.at[c].at[pl.ds(s * RPT + k * C, C)]

    pltpu.async_copy(blk(0), rows0, sem_i0)
    for k in range(NB):
      b = k % 2
      pltpu.make_async_copy(blk(k), rbufs[b], sin[b]).wait()
      if k + 1 < NB:
        pltpu.async_copy(blk(k + 1), rbufs[1 - b], sin[1 - b])
      if k >= 2:
        pltpu.make_async_copy(fbufs[b], oblk(k - 2), sout[b]).wait()

      @pl.loop(0, C)
      def _(i):
        for g in range(D // 32):
          v = rbufs[b][i, pl.ds(32 * g, 32)]
          a, bb = plsc.unpack(v, format=plsc.PackFormat.INTERLEAVED)
          plsc.store_scatter(fbufs[b].at[i], [32 * g + ev_idx], a)
          plsc.store_scatter(fbufs[b].at[i], [32 * g + 1 + ev_idx], bb)

      pltpu.async_copy(fbufs[b], oblk(k), sout[b])
    pltpu.make_async_copy(fbuf, oblk(NB - 2), sem_o0).wait()
    pltpu.make_async_copy(fbuf1, oblk(NB - 1), sem_o1).wait()
    if with_counts:
      pltpu.sync_copy(cnt_sh.at[pl.ds(s * RPT, RPT)], cnt_v)
      pltpu.sync_copy(cnt_v, cnt_out.at[pl.ds(c * NP + s * RPT, RPT)])

  if with_counts:
    def body_wc(table, src_i, dst_i, acc_out, cnt_out, acc_sh, src_v,
                dst_v, rows0, rows1, fbuf, fbuf1, sem_i0, sem_i1,
                sem_o0, sem_o1, sem_g0, sem_g1, sem_s0, sem_s1,
                cnt_sh, ones_v, cnt_v, sem_c0, sem_c1):
      body(table, src_i, dst_i, acc_out, cnt_out, acc_sh, src_v, dst_v,
           rows0, rows1, fbuf, fbuf1, sem_i0, sem_i1, sem_o0, sem_o1,
           sem_g0, sem_g1, sem_s0, sem_s1,
           cnt_sh, ones_v, cnt_v, sem_c0, sem_c1)
    fn = body_wc
  else:
    def body_nc(table, src_i, dst_i, acc_out, acc_sh, src_v, dst_v,
                rows0, rows1, fbuf, fbuf1, sem_i0, sem_i1, sem_o0, sem_o1,
                sem_g0, sem_g1, sem_s0, sem_s1):
      body(table, src_i, dst_i, acc_out, None, acc_sh, src_v, dst_v,
           rows0, rows1, fbuf, fbuf1, sem_i0, sem_i1, sem_o0, sem_o1,
           sem_g0, sem_g1, sem_s0, sem_s1)
    fn = body_nc

  return pl.kernel(
      fn, out_type=out_type, mesh=_mesh, scratch_types=scratch,
      compiler_params=pltpu.CompilerParams(
          use_tc_tiling_on_sc=False, needs_layout_passes=False),
      name="sc_agg_cnt" if with_counts else "sc_agg")


_sc_agg_counts = _make_sc(True)
_sc_agg = _make_sc(False)


def _make_combine(out_dtype):
  def body(acc_ref, cnt_ref, h_ref, wl_ref, wr_ref, b_ref, out_ref):
    agg = acc_ref[0] + acc_ref[1]
    cnt = jnp.sum(cnt_ref[...], axis=0)[:, None]
    mean = agg * (1.0 / jnp.maximum(cnt, 1.0))
    dn = (((1,), (1,)), ((), ()))
    out = (
        lax.dot_general(mean, wl_ref[...], dn,
                        preferred_element_type=jnp.float32)
        + lax.dot_general(h_ref[...].astype(jnp.float32), wr_ref[...], dn,
                          preferred_element_type=jnp.float32)
        + b_ref[...])
    out_ref[...] = out.astype(out_dtype)

  return pl.pallas_call(
      body,
      grid=(NP // BT,),
      in_specs=[
          pl.BlockSpec((NC, BT, D), lambda i: (0, i, 0)),
          pl.BlockSpec((NC, BT), lambda i: (0, i)),
          pl.BlockSpec((BT, D), lambda i: (i, 0)),
          pl.BlockSpec((D, D), lambda i: (0, 0)),
          pl.BlockSpec((D, D), lambda i: (0, 0)),
          pl.BlockSpec((1, D), lambda i: (0, 0)),
      ],
      out_specs=pl.BlockSpec((BT, D), lambda i: (i, 0)),
      out_shape=jax.ShapeDtypeStruct((N, D), out_dtype),
  )


_tc_combine_mid = _make_combine(jnp.bfloat16)
_tc_combine_out = _make_combine(jnp.float32)


def _edge_prep_body(ei_ref, src_ref, dst_ref):
  w = pl.program_id(0)
  e0 = w * (NCH * C)
  eidx = e0 + lax.broadcasted_iota(jnp.int32, (NCH, C), 0) * C + \
      lax.broadcasted_iota(jnp.int32, (NCH, C), 1)
  valid = eidx < E
  s = ei_ref[0].reshape(NCH, C)
  d = ei_ref[1].reshape(NCH, C)
  src_ref[0] = jnp.where(valid, s, eidx % N)
  dst_ref[0] = jnp.where(valid, d, N + eidx % (NP - N))


_edge_prep = pl.pallas_call(
    _edge_prep_body,
    grid=(NW,),
    in_specs=[pl.BlockSpec((2, NCH * C), lambda i: (0, i))],
    out_specs=[
        pl.BlockSpec((1, NCH, C), lambda i: (i, 0, 0)),
        pl.BlockSpec((1, NCH, C), lambda i: (i, 0, 0)),
    ],
    out_shape=[
        jax.ShapeDtypeStruct((NW, NCH, C), jnp.int32),
        jax.ShapeDtypeStruct((NW, NCH, C), jnp.int32),
    ],
)


@jax.jit
def kernel(x, edge_index, W_l0, b_l0, W_r0, W_l1, b_l1, W_r1):
  src, dst = _edge_prep(edge_index)
  x_bf = x.astype(jnp.bfloat16)
  acc1, cnt1 = _sc_agg_counts(x_bf, src, dst)
  cnt1 = cnt1.reshape(NC, NP)
  h1 = _tc_combine_mid(acc1, cnt1, x_bf, W_l0, W_r0, b_l0.reshape(1, D))
  (acc2,) = _sc_agg(h1, src, dst)
  out = _tc_combine_out(acc2, cnt1, h1, W_l1, W_r1, b_l1.reshape(1, D))
  return out


# speedup vs baseline: 2.6193x; 1.0499x over previous
import functools

import jax
import jax.numpy as jnp
from jax import lax
from jax.experimental import pallas as pl
from jax.experimental.pallas import tpu as pltpu
from jax.experimental.pallas import tpu_sc as plsc

N = 10000
E = 320000
D = 128

NC = 2
NS = 16
NW = NC * NS
C = 128
NCH = 80
EPAD = NW * NCH * C
NP = 10240
RPT = NP // NS
BT = 1024

_mesh = plsc.VectorSubcoreMesh(core_axis_name="c", subcore_axis_name="s")


def _make_sc(with_counts: bool):
  out_type = [jax.ShapeDtypeStruct((NC, NP, D), jnp.float32)]
  scratch = [
      pltpu.VMEM_SHARED((NP, D), jnp.bfloat16),
      pltpu.VMEM((NCH, C), jnp.int32),
      pltpu.VMEM((NCH, C), jnp.int32),
      pltpu.VMEM((C, D), jnp.bfloat16),
      pltpu.VMEM((C, D), jnp.bfloat16),
      pltpu.VMEM((C, D), jnp.float32),
      pltpu.VMEM((C, D), jnp.float32),
      pltpu.SemaphoreType.DMA,
      pltpu.SemaphoreType.DMA,
      pltpu.SemaphoreType.DMA,
      pltpu.SemaphoreType.DMA,
      pltpu.SemaphoreType.DMA,
      pltpu.SemaphoreType.DMA,
      pltpu.SemaphoreType.DMA,
      pltpu.SemaphoreType.DMA,
  ]
  if with_counts:
    out_type.append(jax.ShapeDtypeStruct((NC * NP,), jnp.float32))
    scratch += [
        pltpu.VMEM_SHARED((NP,), jnp.float32),
        pltpu.VMEM((C,), jnp.float32),
        pltpu.VMEM((RPT,), jnp.float32),
        pltpu.SemaphoreType.DMA,
        pltpu.SemaphoreType.DMA,
    ]

  def body(table, src_i, dst_i, acc_out, cnt_out, acc_sh, src_v, dst_v,
           rows0, rows1, fbuf, fbuf1, sem_i0, sem_i1, sem_o0, sem_o1,
           sem_g0, sem_g1, sem_s0, sem_s1,
           cnt_sh=None, ones_v=None, cnt_v=None, sem_c0=None, sem_c1=None):
    c = lax.axis_index("c")
    s = lax.axis_index("s")
    w = c * NS + s
    zero16 = jnp.zeros((16,), jnp.float32)
    zero32 = jnp.zeros((32,), jnp.bfloat16)

    @pl.loop(0, C)
    def _(i):
      for k in range(D // 32):
        rows0[i, pl.ds(32 * k, 32)] = zero32

    @pl.loop(0, RPT // C)
    def _(k):
      pltpu.sync_copy(rows0, acc_sh.at[pl.ds(s * RPT + k * C, C)])

    pltpu.sync_copy(src_i.at[w], src_v)
    pltpu.sync_copy(dst_i.at[w], dst_v)
    if with_counts:
      @pl.loop(0, RPT // 16)
      def _(i):
        cnt_v[pl.ds(16 * i, 16)] = zero16
      pltpu.sync_copy(cnt_v, cnt_sh.at[pl.ds(s * RPT, RPT)])
      one16 = jnp.ones((16,), jnp.float32)
      for k in range(C // 16):
        ones_v[pl.ds(16 * k, 16)] = one16
    plsc.subcore_barrier()

    pltpu.async_copy(table.at[src_v.at[0]], rows0, sem_g0)
    pltpu.async_copy(table.at[src_v.at[1]], rows1, sem_g1)

    @pl.loop(0, NCH, step=2)
    def _(j):
      pltpu.make_async_copy(table.at[src_v.at[j]], rows0, sem_g0).wait()
      pltpu.sync_copy(rows0, acc_sh.at[dst_v.at[j]], add=True)
      if with_counts:
        pltpu.sync_copy(ones_v, cnt_sh.at[dst_v.at[j]], add=True)
      pltpu.async_copy(table.at[src_v.at[(j + 2) % NCH]], rows0, sem_g0)

      j1 = j + 1
      pltpu.make_async_copy(table.at[src_v.at[j1]], rows1, sem_g1).wait()
      pltpu.sync_copy(rows1, acc_sh.at[dst_v.at[j1]], add=True)
      if with_counts:
        pltpu.sync_copy(ones_v, cnt_sh.at[dst_v.at[j1]], add=True)
      pltpu.async_copy(table.at[src_v.at[(j1 + 2) % NCH]], rows1, sem_g1)

    pltpu.make_async_copy(table.at[src_v.at[0]], rows0, sem_g0).wait()
    pltpu.make_async_copy(table.at[src_v.at[1]], rows1, sem_g1).wait()
    plsc.subcore_barrier()

    ev_idx = 2 * lax.iota(jnp.int32, 16)
    NB = RPT // C
    rbufs = (rows0, rows1)
    fbufs = (fbuf, fbuf1)
    sin = (sem_i0, sem_i1)
    sout = (sem_o0, sem_o1)

    def blk(k):
      return acc_sh.at[pl.ds(s * RPT + k * C, C)]

    def oblk(k):
      return acc_out.at[c].at[pl.ds(s * RPT + k * C, C)]

    pltpu.async_copy(blk(0), rows0, sem_i0)
    for k in range(NB):
      b = k % 2
      pltpu.make_async_copy(blk(k), rbufs[b], sin[b]).wait()
      if k + 1 < NB:
        pltpu.async_copy(blk(k + 1), rbufs[1 - b], sin[1 - b])
      if k >= 2:
        pltpu.make_async_copy(fbufs[b], oblk(k - 2), sout[b]).wait()

      @pl.loop(0, C)
      def _(i):
        for g in range(D // 32):
          v = rbufs[b][i, pl.ds(32 * g, 32)]
          a, bb = plsc.unpack(v, format=plsc.PackFormat.INTERLEAVED)
          plsc.store_scatter(fbufs[b].at[i], [32 * g + ev_idx], a)
          plsc.store_scatter(fbufs[b].at[i], [32 * g + 1 + ev_idx], bb)

      pltpu.async_copy(fbufs[b], oblk(k), sout[b])
    pltpu.make_async_copy(fbuf, oblk(NB - 2), sem_o0).wait()
    pltpu.make_async_copy(fbuf1, oblk(NB - 1), sem_o1).wait()
    if with_counts:
      pltpu.sync_copy(cnt_sh.at[pl.ds(s * RPT, RPT)], cnt_v)
      pltpu.sync_copy(cnt_v, cnt_out.at[pl.ds(c * NP + s * RPT, RPT)])

  if with_counts:
    def body_wc(table, src_i, dst_i, acc_out, cnt_out, acc_sh, src_v,
                dst_v, rows0, rows1, fbuf, fbuf1, sem_i0, sem_i1,
                sem_o0, sem_o1, sem_g0, sem_g1, sem_s0, sem_s1,
                cnt_sh, ones_v, cnt_v, sem_c0, sem_c1):
      body(table, src_i, dst_i, acc_out, cnt_out, acc_sh, src_v, dst_v,
           rows0, rows1, fbuf, fbuf1, sem_i0, sem_i1, sem_o0, sem_o1,
           sem_g0, sem_g1, sem_s0, sem_s1,
           cnt_sh, ones_v, cnt_v, sem_c0, sem_c1)
    fn = body_wc
  else:
    def body_nc(table, src_i, dst_i, acc_out, acc_sh, src_v, dst_v,
                rows0, rows1, fbuf, fbuf1, sem_i0, sem_i1, sem_o0, sem_o1,
                sem_g0, sem_g1, sem_s0, sem_s1):
      body(table, src_i, dst_i, acc_out, None, acc_sh, src_v, dst_v,
           rows0, rows1, fbuf, fbuf1, sem_i0, sem_i1, sem_o0, sem_o1,
           sem_g0, sem_g1, sem_s0, sem_s1)
    fn = body_nc

  return pl.kernel(
      fn, out_type=out_type, mesh=_mesh, scratch_types=scratch,
      compiler_params=pltpu.CompilerParams(
          use_tc_tiling_on_sc=False, needs_layout_passes=False),
      name="sc_agg_cnt" if with_counts else "sc_agg")


_sc_agg_counts = _make_sc(True)
_sc_agg = _make_sc(False)


def _make_combine(out_dtype):
  def body(acc_ref, cnt_ref, h_ref, wl_ref, wr_ref, b_ref, out_ref):
    agg = acc_ref[0] + acc_ref[1]
    cnt = jnp.sum(cnt_ref[...], axis=0)[:, None]
    mean = agg * (1.0 / jnp.maximum(cnt, 1.0))
    dn = (((1,), (1,)), ((), ()))
    out = (
        lax.dot_general(mean, wl_ref[...], dn,
                        preferred_element_type=jnp.float32)
        + lax.dot_general(h_ref[...].astype(jnp.float32), wr_ref[...], dn,
                          preferred_element_type=jnp.float32)
        + b_ref[...])
    out_ref[...] = out.astype(out_dtype)

  return pl.pallas_call(
      body,
      grid=(NP // BT,),
      in_specs=[
          pl.BlockSpec((NC, BT, D), lambda i: (0, i, 0)),
          pl.BlockSpec((NC, BT), lambda i: (0, i)),
          pl.BlockSpec((BT, D), lambda i: (i, 0)),
          pl.BlockSpec((D, D), lambda i: (0, 0)),
          pl.BlockSpec((D, D), lambda i: (0, 0)),
          pl.BlockSpec((1, D), lambda i: (0, 0)),
      ],
      out_specs=pl.BlockSpec((BT, D), lambda i: (i, 0)),
      out_shape=jax.ShapeDtypeStruct((N, D), out_dtype),
  )


_tc_combine_mid = _make_combine(jnp.bfloat16)
_tc_combine_out = _make_combine(jnp.float32)


_WPB = 4


def _edge_prep_body(ei_ref, src_ref, dst_ref):
  g = pl.program_id(0)
  e0 = g * (_WPB * NCH * C)
  shp = (_WPB, NCH, C)
  eidx = e0 + (lax.broadcasted_iota(jnp.int32, shp, 0) * (NCH * C)
               + lax.broadcasted_iota(jnp.int32, shp, 1) * C
               + lax.broadcasted_iota(jnp.int32, shp, 2))
  valid = eidx < E
  s = ei_ref[0].reshape(shp)
  d = ei_ref[1].reshape(shp)
  src_ref[...] = jnp.where(valid, s, eidx % N)
  dst_ref[...] = jnp.where(valid, d, N + eidx % (NP - N))


_edge_prep = pl.pallas_call(
    _edge_prep_body,
    grid=(NW // _WPB,),
    in_specs=[pl.BlockSpec((2, _WPB * NCH * C), lambda i: (0, i))],
    out_specs=[
        pl.BlockSpec((_WPB, NCH, C), lambda i: (i, 0, 0)),
        pl.BlockSpec((_WPB, NCH, C), lambda i: (i, 0, 0)),
    ],
    out_shape=[
        jax.ShapeDtypeStruct((NW, NCH, C), jnp.int32),
        jax.ShapeDtypeStruct((NW, NCH, C), jnp.int32),
    ],
)


@jax.jit
def kernel(x, edge_index, W_l0, b_l0, W_r0, W_l1, b_l1, W_r1):
  src, dst = _edge_prep(edge_index)
  x_bf = x.astype(jnp.bfloat16)
  acc1, cnt1 = _sc_agg_counts(x_bf, src, dst)
  cnt1 = cnt1.reshape(NC, NP)
  h1 = _tc_combine_mid(acc1, cnt1, x_bf, W_l0, W_r0, b_l0.reshape(1, D))
  (acc2,) = _sc_agg(h1, src, dst)
  out = _tc_combine_out(acc2, cnt1, h1, W_l1, W_r1, b_l1.reshape(1, D))
  return out


# speedup vs baseline: 2.6447x; 1.0097x over previous
import functools

import jax
import jax.numpy as jnp
from jax import lax
from jax.experimental import pallas as pl
from jax.experimental.pallas import tpu as pltpu
from jax.experimental.pallas import tpu_sc as plsc

N = 10000
E = 320000
D = 128

NC = 2
NS = 16
NW = NC * NS
C = 128
NCH = 80
EPAD = NW * NCH * C
NP = 10240
RPT = NP // NS
BT = 1024

_mesh = plsc.VectorSubcoreMesh(core_axis_name="c", subcore_axis_name="s")


def _make_sc(with_counts: bool):
  out_type = [jax.ShapeDtypeStruct((NC, NP, D), jnp.float32)]
  scratch = [
      pltpu.VMEM_SHARED((NP, D), jnp.bfloat16),
      pltpu.VMEM((NCH, C), jnp.int32),
      pltpu.VMEM((NCH, C), jnp.int32),
      pltpu.VMEM((C, D), jnp.bfloat16),
      pltpu.VMEM((C, D), jnp.bfloat16),
      pltpu.VMEM((C, D), jnp.float32),
      pltpu.VMEM((C, D), jnp.float32),
      pltpu.SemaphoreType.DMA,
      pltpu.SemaphoreType.DMA,
      pltpu.SemaphoreType.DMA,
      pltpu.SemaphoreType.DMA,
      pltpu.SemaphoreType.DMA,
      pltpu.SemaphoreType.DMA,
      pltpu.SemaphoreType.DMA,
      pltpu.SemaphoreType.DMA,
  ]
  if with_counts:
    out_type.append(jax.ShapeDtypeStruct((NC * NP,), jnp.float32))
    scratch += [
        pltpu.VMEM_SHARED((NP,), jnp.float32),
        pltpu.VMEM((C,), jnp.float32),
        pltpu.VMEM((RPT,), jnp.float32),
        pltpu.SemaphoreType.DMA,
        pltpu.SemaphoreType.DMA,
    ]

  def body(table, src_i, dst_i, acc_out, cnt_out, acc_sh, src_v, dst_v,
           rows0, rows1, fbuf, fbuf1, sem_i0, sem_i1, sem_o0, sem_o1,
           sem_g0, sem_g1, sem_s0, sem_s1,
           cnt_sh=None, ones_v=None, cnt_v=None, sem_c0=None, sem_c1=None):
    c = lax.axis_index("c")
    s = lax.axis_index("s")
    w = c * NS + s
    zero16 = jnp.zeros((16,), jnp.float32)
    zero32 = jnp.zeros((32,), jnp.bfloat16)

    @pl.loop(0, C)
    def _(i):
      for k in range(D // 32):
        rows0[i, pl.ds(32 * k, 32)] = zero32

    @pl.loop(0, RPT // C)
    def _(k):
      pltpu.sync_copy(rows0, acc_sh.at[pl.ds(s * RPT + k * C, C)])

    pltpu.sync_copy(src_i.at[w], src_v)
    pltpu.sync_copy(dst_i.at[w], dst_v)
    if with_counts:
      @pl.loop(0, RPT // 16)
      def _(i):
        cnt_v[pl.ds(16 * i, 16)] = zero16
      pltpu.sync_copy(cnt_v, cnt_sh.at[pl.ds(s * RPT, RPT)])
      one16 = jnp.ones((16,), jnp.float32)
      for k in range(C // 16):
        ones_v[pl.ds(16 * k, 16)] = one16
    plsc.subcore_barrier()

    pltpu.async_copy(table.at[src_v.at[0]], rows0, sem_g0)
    pltpu.async_copy(table.at[src_v.at[1]], rows1, sem_g1)

    @pl.loop(0, NCH, step=2)
    def _(j):
      pltpu.make_async_copy(table.at[src_v.at[j]], rows0, sem_g0).wait()
      pltpu.sync_copy(rows0, acc_sh.at[dst_v.at[j]], add=True)
      if with_counts:
        pltpu.sync_copy(ones_v, cnt_sh.at[dst_v.at[j]], add=True)
      pltpu.async_copy(table.at[src_v.at[(j + 2) % NCH]], rows0, sem_g0)

      j1 = j + 1
      pltpu.make_async_copy(table.at[src_v.at[j1]], rows1, sem_g1).wait()
      pltpu.sync_copy(rows1, acc_sh.at[dst_v.at[j1]], add=True)
      if with_counts:
        pltpu.sync_copy(ones_v, cnt_sh.at[dst_v.at[j1]], add=True)
      pltpu.async_copy(table.at[src_v.at[(j1 + 2) % NCH]], rows1, sem_g1)

    pltpu.make_async_copy(table.at[src_v.at[0]], rows0, sem_g0).wait()
    pltpu.make_async_copy(table.at[src_v.at[1]], rows1, sem_g1).wait()
    plsc.subcore_barrier()

    ev_idx = 2 * lax.iota(jnp.int32, 16)
    NB = RPT // C
    rbufs = (rows0, rows1)
    fbufs = (fbuf, fbuf1)
    sin = (sem_i0, sem_i1)
    sout = (sem_o0, sem_o1)

    def blk(k):
      return acc_sh.at[pl.ds(s * RPT + k * C, C)]

    def oblk(k):
      return acc_out.at[c].at[pl.ds(s * RPT + k * C, C)]

    pltpu.async_copy(blk(0), rows0, sem_i0)
    for k in range(NB):
      b = k % 2
      pltpu.make_async_copy(blk(k), rbufs[b], sin[b]).wait()
      if k + 1 < NB:
        pltpu.async_copy(blk(k + 1), rbufs[1 - b], sin[1 - b])
      if k >= 2:
        pltpu.make_async_copy(fbufs[b], oblk(k - 2), sout[b]).wait()

      @pl.loop(0, C)
      def _(i):
        for g in range(D // 32):
          v = rbufs[b][i, pl.ds(32 * g, 32)]
          a, bb = plsc.unpack(v, format=plsc.PackFormat.INTERLEAVED)
          plsc.store_scatter(fbufs[b].at[i], [32 * g + ev_idx], a)
          plsc.store_scatter(fbufs[b].at[i], [32 * g + 1 + ev_idx], bb)

      pltpu.async_copy(fbufs[b], oblk(k), sout[b])
    pltpu.make_async_copy(fbuf, oblk(NB - 2), sem_o0).wait()
    pltpu.make_async_copy(fbuf1, oblk(NB - 1), sem_o1).wait()
    if with_counts:
      pltpu.sync_copy(cnt_sh.at[pl.ds(s * RPT, RPT)], cnt_v)
      pltpu.sync_copy(cnt_v, cnt_out.at[pl.ds(c * NP + s * RPT, RPT)])

  if with_counts:
    def body_wc(table, src_i, dst_i, acc_out, cnt_out, acc_sh, src_v,
                dst_v, rows0, rows1, fbuf, fbuf1, sem_i0, sem_i1,
                sem_o0, sem_o1, sem_g0, sem_g1, sem_s0, sem_s1,
                cnt_sh, ones_v, cnt_v, sem_c0, sem_c1):
      body(table, src_i, dst_i, acc_out, cnt_out, acc_sh, src_v, dst_v,
           rows0, rows1, fbuf, fbuf1, sem_i0, sem_i1, sem_o0, sem_o1,
           sem_g0, sem_g1, sem_s0, sem_s1,
           cnt_sh, ones_v, cnt_v, sem_c0, sem_c1)
    fn = body_wc
  else:
    def body_nc(table, src_i, dst_i, acc_out, acc_sh, src_v, dst_v,
                rows0, rows1, fbuf, fbuf1, sem_i0, sem_i1, sem_o0, sem_o1,
                sem_g0, sem_g1, sem_s0, sem_s1):
      body(table, src_i, dst_i, acc_out, None, acc_sh, src_v, dst_v,
           rows0, rows1, fbuf, fbuf1, sem_i0, sem_i1, sem_o0, sem_o1,
           sem_g0, sem_g1, sem_s0, sem_s1)
    fn = body_nc

  return pl.kernel(
      fn, out_type=out_type, mesh=_mesh, scratch_types=scratch,
      compiler_params=pltpu.CompilerParams(
          use_tc_tiling_on_sc=False, needs_layout_passes=False),
      name="sc_agg_cnt" if with_counts else "sc_agg")


_sc_agg_counts = _make_sc(True)
_sc_agg = _make_sc(False)


def _make_combine(out_dtype):
  def body(acc_ref, cnt_ref, h_ref, wl_ref, wr_ref, b_ref, out_ref):
    agg = acc_ref[0] + acc_ref[1]
    cnt = jnp.sum(cnt_ref[...], axis=0)[:, None]
    mean = agg * (1.0 / jnp.maximum(cnt, 1.0))
    dn = (((1,), (1,)), ((), ()))
    out = (
        lax.dot_general(mean, wl_ref[...], dn,
                        preferred_element_type=jnp.float32)
        + lax.dot_general(h_ref[...].astype(jnp.float32), wr_ref[...], dn,
                          preferred_element_type=jnp.float32)
        + b_ref[...])
    out_ref[...] = out.astype(out_dtype)

  return pl.pallas_call(
      body,
      grid=(NP // BT,),
      in_specs=[
          pl.BlockSpec((NC, BT, D), lambda i: (0, i, 0)),
          pl.BlockSpec((NC, BT), lambda i: (0, i)),
          pl.BlockSpec((BT, D), lambda i: (i, 0)),
          pl.BlockSpec((D, D), lambda i: (0, 0)),
          pl.BlockSpec((D, D), lambda i: (0, 0)),
          pl.BlockSpec((1, D), lambda i: (0, 0)),
      ],
      out_specs=pl.BlockSpec((BT, D), lambda i: (i, 0)),
      out_shape=jax.ShapeDtypeStruct((N, D), out_dtype),
  )


_tc_combine_mid = _make_combine(jnp.bfloat16)
_tc_combine_out = _make_combine(jnp.float32)


_WPB = 8


def _edge_prep_body(ei_ref, src_ref, dst_ref):
  g = pl.program_id(0)
  e0 = g * (_WPB * NCH * C)
  shp = (_WPB, NCH, C)
  eidx = e0 + (lax.broadcasted_iota(jnp.int32, shp, 0) * (NCH * C)
               + lax.broadcasted_iota(jnp.int32, shp, 1) * C
               + lax.broadcasted_iota(jnp.int32, shp, 2))
  valid = eidx < E
  s = ei_ref[0].reshape(shp)
  d = ei_ref[1].reshape(shp)
  src_ref[...] = jnp.where(valid, s, eidx % N)
  dst_ref[...] = jnp.where(valid, d, N + eidx % (NP - N))


_edge_prep = pl.pallas_call(
    _edge_prep_body,
    grid=(NW // _WPB,),
    in_specs=[pl.BlockSpec((2, _WPB * NCH * C), lambda i: (0, i))],
    out_specs=[
        pl.BlockSpec((_WPB, NCH, C), lambda i: (i, 0, 0)),
        pl.BlockSpec((_WPB, NCH, C), lambda i: (i, 0, 0)),
    ],
    out_shape=[
        jax.ShapeDtypeStruct((NW, NCH, C), jnp.int32),
        jax.ShapeDtypeStruct((NW, NCH, C), jnp.int32),
    ],
)


@jax.jit
def kernel(x, edge_index, W_l0, b_l0, W_r0, W_l1, b_l1, W_r1):
  src, dst = _edge_prep(edge_index)
  x_bf = x.astype(jnp.bfloat16)
  acc1, cnt1 = _sc_agg_counts(x_bf, src, dst)
  cnt1 = cnt1.reshape(NC, NP)
  h1 = _tc_combine_mid(acc1, cnt1, x_bf, W_l0, W_r0, b_l0.reshape(1, D))
  (acc2,) = _sc_agg(h1, src, dst)
  out = _tc_combine_out(acc2, cnt1, h1, W_l1, W_r1, b_l1.reshape(1, D))
  return out


# speedup vs baseline: 2.6507x; 1.0023x over previous
import functools

import jax
import jax.numpy as jnp
from jax import lax
from jax.experimental import pallas as pl
from jax.experimental.pallas import tpu as pltpu
from jax.experimental.pallas import tpu_sc as plsc

N = 10000
E = 320000
D = 128

NC = 2
NS = 16
NW = NC * NS
C = 128
NCH = 80
EPAD = NW * NCH * C
NP = 10240
RPT = NP // NS
BT = 1024

_mesh = plsc.VectorSubcoreMesh(core_axis_name="c", subcore_axis_name="s")


def _make_sc(with_counts: bool):
  out_type = [jax.ShapeDtypeStruct((NC, NP, D), jnp.float32)]
  scratch = [
      pltpu.VMEM_SHARED((NP, D), jnp.bfloat16),
      pltpu.VMEM((NCH, C), jnp.int32),
      pltpu.VMEM((NCH, C), jnp.int32),
      pltpu.VMEM((C, D), jnp.bfloat16),
      pltpu.VMEM((C, D), jnp.bfloat16),
      pltpu.VMEM((C, D), jnp.float32),
      pltpu.VMEM((C, D), jnp.float32),
      pltpu.SemaphoreType.DMA,
      pltpu.SemaphoreType.DMA,
      pltpu.SemaphoreType.DMA,
      pltpu.SemaphoreType.DMA,
      pltpu.SemaphoreType.DMA,
      pltpu.SemaphoreType.DMA,
      pltpu.SemaphoreType.DMA,
      pltpu.SemaphoreType.DMA,
  ]
  if with_counts:
    out_type.append(jax.ShapeDtypeStruct((NC * NP,), jnp.float32))
    scratch += [
        pltpu.VMEM_SHARED((NP,), jnp.float32),
        pltpu.VMEM((C,), jnp.float32),
        pltpu.VMEM((RPT,), jnp.float32),
        pltpu.SemaphoreType.DMA,
        pltpu.SemaphoreType.DMA,
    ]

  def body(table, src_i, dst_i, acc_out, cnt_out, acc_sh, src_v, dst_v,
           rows0, rows1, fbuf, fbuf1, sem_i0, sem_i1, sem_o0, sem_o1,
           sem_g0, sem_g1, sem_s0, sem_s1,
           cnt_sh=None, ones_v=None, cnt_v=None, sem_c0=None, sem_c1=None):
    c = lax.axis_index("c")
    s = lax.axis_index("s")
    w = c * NS + s
    zero16 = jnp.zeros((16,), jnp.float32)
    zero32 = jnp.zeros((32,), jnp.bfloat16)

    @pl.loop(0, C)
    def _(i):
      for k in range(D // 32):
        rows0[i, pl.ds(32 * k, 32)] = zero32

    @pl.loop(0, RPT // C)
    def _(k):
      pltpu.sync_copy(rows0, acc_sh.at[pl.ds(s * RPT + k * C, C)])

    pltpu.sync_copy(src_i.at[w], src_v)
    pltpu.sync_copy(dst_i.at[w], dst_v)
    if with_counts:
      @pl.loop(0, RPT // 16)
      def _(i):
        cnt_v[pl.ds(16 * i, 16)] = zero16
      pltpu.sync_copy(cnt_v, cnt_sh.at[pl.ds(s * RPT, RPT)])
      one16 = jnp.ones((16,), jnp.float32)
      for k in range(C // 16):
        ones_v[pl.ds(16 * k, 16)] = one16
    plsc.subcore_barrier()

    pltpu.async_copy(table.at[src_v.at[0]], rows0, sem_g0)
    pltpu.async_copy(table.at[src_v.at[1]], rows1, sem_g1)

    @pl.loop(0, NCH, step=2)
    def _(j):
      pltpu.make_async_copy(table.at[src_v.at[j]], rows0, sem_g0).wait()
      pltpu.sync_copy(rows0, acc_sh.at[dst_v.at[j]], add=True)
      if with_counts:
        pltpu.sync_copy(ones_v, cnt_sh.at[dst_v.at[j]], add=True)
      pltpu.async_copy(table.at[src_v.at[(j + 2) % NCH]], rows0, sem_g0)

      j1 = j + 1
      pltpu.make_async_copy(table.at[src_v.at[j1]], rows1, sem_g1).wait()
      pltpu.sync_copy(rows1, acc_sh.at[dst_v.at[j1]], add=True)
      if with_counts:
        pltpu.sync_copy(ones_v, cnt_sh.at[dst_v.at[j1]], add=True)
      pltpu.async_copy(table.at[src_v.at[(j1 + 2) % NCH]], rows1, sem_g1)

    pltpu.make_async_copy(table.at[src_v.at[0]], rows0, sem_g0).wait()
    pltpu.make_async_copy(table.at[src_v.at[1]], rows1, sem_g1).wait()
    plsc.subcore_barrier()

    ev_idx = 2 * lax.iota(jnp.int32, 16)
    NB = RPT // C
    rbufs = (rows0, rows1)
    fbufs = (fbuf, fbuf1)
    sin = (sem_i0, sem_i1)
    sout = (sem_o0, sem_o1)

    def blk(k):
      return acc_sh.at[pl.ds(s * RPT + k * C, C)]

    def oblk(k):
      return acc_out.at[c].at[pl.ds(s * RPT + k * C, C)]

    pltpu.async_copy(blk(0), rows0, sem_i0)
    for k in range(NB):
      b = k % 2
      pltpu.make_async_copy(blk(k), rbufs[b], sin[b]).wait()
      if k + 1 < NB:
        pltpu.async_copy(blk(k + 1), rbufs[1 - b], sin[1 - b])
      if k >= 2:
        pltpu.make_async_copy(fbufs[b], oblk(k - 2), sout[b]).wait()

      @pl.loop(0, C)
      def _(i):
        for g in range(D // 32):
          v = rbufs[b][i, pl.ds(32 * g, 32)]
          a, bb = plsc.unpack(v, format=plsc.PackFormat.INTERLEAVED)
          plsc.store_scatter(fbufs[b].at[i], [32 * g + ev_idx], a)
          plsc.store_scatter(fbufs[b].at[i], [32 * g + 1 + ev_idx], bb)

      pltpu.async_copy(fbufs[b], oblk(k), sout[b])
    pltpu.make_async_copy(fbuf, oblk(NB - 2), sem_o0).wait()
    pltpu.make_async_copy(fbuf1, oblk(NB - 1), sem_o1).wait()
    if with_counts:
      pltpu.sync_copy(cnt_sh.at[pl.ds(s * RPT, RPT)], cnt_v)
      pltpu.sync_copy(cnt_v, cnt_out.at[pl.ds(c * NP + s * RPT, RPT)])

  if with_counts:
    def body_wc(table, src_i, dst_i, acc_out, cnt_out, acc_sh, src_v,
                dst_v, rows0, rows1, fbuf, fbuf1, sem_i0, sem_i1,
                sem_o0, sem_o1, sem_g0, sem_g1, sem_s0, sem_s1,
                cnt_sh, ones_v, cnt_v, sem_c0, sem_c1):
      body(table, src_i, dst_i, acc_out, cnt_out, acc_sh, src_v, dst_v,
           rows0, rows1, fbuf, fbuf1, sem_i0, sem_i1, sem_o0, sem_o1,
           sem_g0, sem_g1, sem_s0, sem_s1,
           cnt_sh, ones_v, cnt_v, sem_c0, sem_c1)
    fn = body_wc
  else:
    def body_nc(table, src_i, dst_i, acc_out, acc_sh, src_v, dst_v,
                rows0, rows1, fbuf, fbuf1, sem_i0, sem_i1, sem_o0, sem_o1,
                sem_g0, sem_g1, sem_s0, sem_s1):
      body(table, src_i, dst_i, acc_out, None, acc_sh, src_v, dst_v,
           rows0, rows1, fbuf, fbuf1, sem_i0, sem_i1, sem_o0, sem_o1,
           sem_g0, sem_g1, sem_s0, sem_s1)
    fn = body_nc

  return pl.kernel(
      fn, out_type=out_type, mesh=_mesh, scratch_types=scratch,
      compiler_params=pltpu.CompilerParams(
          use_tc_tiling_on_sc=False, needs_layout_passes=False),
      name="sc_agg_cnt" if with_counts else "sc_agg")


_sc_agg_counts = _make_sc(True)
_sc_agg = _make_sc(False)


def _make_combine(out_dtype):
  def body(acc_ref, cnt_ref, h_ref, wl_ref, wr_ref, b_ref, out_ref):
    agg = acc_ref[0] + acc_ref[1]
    cnt = jnp.sum(cnt_ref[...], axis=0)[:, None]
    mean = agg * (1.0 / jnp.maximum(cnt, 1.0))
    dn = (((1,), (1,)), ((), ()))
    out = (
        lax.dot_general(mean, wl_ref[...], dn,
                        preferred_element_type=jnp.float32)
        + lax.dot_general(h_ref[...].astype(jnp.float32), wr_ref[...], dn,
                          preferred_element_type=jnp.float32)
        + b_ref[...])
    out_ref[...] = out.astype(out_dtype)

  return pl.pallas_call(
      body,
      grid=(NP // BT,),
      in_specs=[
          pl.BlockSpec((NC, BT, D), lambda i: (0, i, 0)),
          pl.BlockSpec((NC, BT), lambda i: (0, i)),
          pl.BlockSpec((BT, D), lambda i: (i, 0)),
          pl.BlockSpec((D, D), lambda i: (0, 0)),
          pl.BlockSpec((D, D), lambda i: (0, 0)),
          pl.BlockSpec((1, D), lambda i: (0, 0)),
      ],
      out_specs=pl.BlockSpec((BT, D), lambda i: (i, 0)),
      out_shape=jax.ShapeDtypeStruct((N, D), out_dtype),
  )


_tc_combine_mid = _make_combine(jnp.bfloat16)
_tc_combine_out = _make_combine(jnp.float32)


_WPB = 16


def _edge_prep_body(ei_ref, src_ref, dst_ref):
  g = pl.program_id(0)
  e0 = g * (_WPB * NCH * C)
  shp = (_WPB, NCH, C)
  eidx = e0 + (lax.broadcasted_iota(jnp.int32, shp, 0) * (NCH * C)
               + lax.broadcasted_iota(jnp.int32, shp, 1) * C
               + lax.broadcasted_iota(jnp.int32, shp, 2))
  valid = eidx < E
  s = ei_ref[0].reshape(shp)
  d = ei_ref[1].reshape(shp)
  src_ref[...] = jnp.where(valid, s, eidx % N)
  dst_ref[...] = jnp.where(valid, d, N + eidx % (NP - N))


_edge_prep = pl.pallas_call(
    _edge_prep_body,
    grid=(NW // _WPB,),
    in_specs=[pl.BlockSpec((2, _WPB * NCH * C), lambda i: (0, i))],
    out_specs=[
        pl.BlockSpec((_WPB, NCH, C), lambda i: (i, 0, 0)),
        pl.BlockSpec((_WPB, NCH, C), lambda i: (i, 0, 0)),
    ],
    out_shape=[
        jax.ShapeDtypeStruct((NW, NCH, C), jnp.int32),
        jax.ShapeDtypeStruct((NW, NCH, C), jnp.int32),
    ],
)


@jax.jit
def kernel(x, edge_index, W_l0, b_l0, W_r0, W_l1, b_l1, W_r1):
  src, dst = _edge_prep(edge_index)
  x_bf = x.astype(jnp.bfloat16)
  acc1, cnt1 = _sc_agg_counts(x_bf, src, dst)
  cnt1 = cnt1.reshape(NC, NP)
  h1 = _tc_combine_mid(acc1, cnt1, x_bf, W_l0, W_r0, b_l0.reshape(1, D))
  (acc2,) = _sc_agg(h1, src, dst)
  out = _tc_combine_out(acc2, cnt1, h1, W_l1, W_r1, b_l1.reshape(1, D))
  return out
